# baseline (speedup 1.0000x reference)
import functools

import jax
import jax.numpy as jnp
from jax import lax
from jax.experimental import pallas as pl
from jax.experimental.pallas import tpu as pltpu

N = 10000
E = 320000
H = 128
G = 64
NPAD = 10240


def _mm_kernel(x_ref, w_ref, b_ref, o_ref, *, act):
    y = jnp.dot(x_ref[...], w_ref[...], preferred_element_type=jnp.float32)
    y = y + b_ref[...]
    if act == "relu":
        y = jnp.maximum(y, 0.0)
    o_ref[...] = y


def _matmul(x, w, b, act=None, block=512):
    m, k = x.shape
    n = w.shape[1]
    assert m % block == 0
    grid = (m // block,)
    return pl.pallas_call(
        functools.partial(_mm_kernel, act=act),
        grid=grid,
        in_specs=[
            pl.BlockSpec((block, k), lambda i: (i, 0)),
            pl.BlockSpec((k, n), lambda i: (0, 0)),
            pl.BlockSpec((1, n), lambda i: (0, 0)),
        ],
        out_specs=pl.BlockSpec((block, n), lambda i: (i, 0)),
        out_shape=jax.ShapeDtypeStruct((m, n), jnp.float32),
    )(x, w, b.reshape(1, -1))


def _gcn(x, ei, W, b, ew):
    n = x.shape[0]
    loop = jnp.arange(n, dtype=ei.dtype)
    src = jnp.concatenate([ei[0], loop])
    dst = jnp.concatenate([ei[1], loop])
    w = jnp.concatenate([ew, jnp.ones((n,), x.dtype)])
    deg = jnp.zeros((n,), x.dtype).at[dst].add(w)
    dis = deg ** -0.5
    norm = dis[src] * dis[dst] * w
    xw = _matmul(jnp.pad(x, ((0, NPAD - n), (0, 0))), W, b * 0.0)[:n]
    out = jnp.zeros((n, W.shape[1]), x.dtype).at[dst].add(xw[src] * norm[:, None])
    return out + b


def _edge_conv(x, ei, W1, b1, W2, b2, em):
    n = x.shape[0]
    src, dst = ei[0], ei[1]
    W1a = W1[:H]
    W1b = W1[H:]
    a = _matmul(jnp.pad(x, ((0, NPAD - n), (0, 0))), W1a - W1b, b1)[:n]
    bb = _matmul(jnp.pad(x, ((0, NPAD - n), (0, 0))), W1b, b1 * 0.0)[:n]
    pe = jax.nn.relu(a[dst] + bb[src])
    h = _matmul(pe, W2, b2)
    h = jnp.where(em[:, None], h, -jnp.inf)
    out = jnp.full((n, h.shape[1]), -jnp.inf, x.dtype).at[dst].max(h)
    return jnp.where(jnp.isneginf(out), 0.0, out)


def _select_mask(score, batch, mask, tie):
    n = score.shape[0]
    neg = jnp.where(mask, score, -jnp.inf)
    order = jnp.lexsort((tie, -neg, batch))
    total = jax.ops.segment_sum(jnp.ones((n,), jnp.int32), batch, num_segments=G)
    offs = jnp.cumsum(total) - total
    cnt = jax.ops.segment_sum(mask.astype(jnp.int32), batch, num_segments=G)
    k = (4 * cnt + 4) // 5
    bs = batch[order]
    rank = jnp.arange(n, dtype=jnp.int32) - offs[bs]
    sel = (rank < k[bs]) & mask[order]
    new_mask = jnp.zeros((n,), jnp.bool_).at[order].set(sel)
    pos = jnp.zeros((n,), jnp.int32).at[order].set(jnp.arange(n, dtype=jnp.int32))
    return new_mask, pos


def kernel(x, edge_index, batch, gcn1_W, gcn1_b, ec1_W1, ec1_b1, ec1_W2, ec1_b2,
           pool1_p, gcn2_W, gcn2_b, ec2_W1, ec2_b1, ec2_W2, ec2_b2, pool2_p,
           fc_W, fc_b):
    n = x.shape[0]
    e = edge_index.shape[1]
    h = jax.nn.relu(_gcn(x, edge_index, gcn1_W, gcn1_b, jnp.ones((e,), x.dtype)))
    h = jax.nn.relu(_edge_conv(h, edge_index, ec1_W1, ec1_b1, ec1_W2, ec1_b2,
                               jnp.ones((e,), jnp.bool_)))
    s1 = jnp.tanh((h @ pool1_p) / jnp.linalg.norm(pool1_p))
    m0 = jnp.ones((n,), jnp.bool_)
    m1, pos1 = _select_mask(s1, batch, m0, jnp.arange(n, dtype=jnp.int32))
    em1 = m1[edge_index[0]] & m1[edge_index[1]]
    h = h * s1[:, None]
    h = jax.nn.relu(_gcn(h, edge_index, gcn2_W, gcn2_b, em1.astype(x.dtype)))
    h = jax.nn.relu(_edge_conv(h, edge_index, ec2_W1, ec2_b1, ec2_W2, ec2_b2, em1))
    s2 = jnp.tanh((h @ pool2_p) / jnp.linalg.norm(pool2_p))
    m2, _ = _select_mask(s2, batch, m1, pos1)
    h = h * s2[:, None]
    hs = jnp.where(m2[:, None], h, 0.0)
    ssum = jax.ops.segment_sum(hs, batch, num_segments=G)
    cnt = jax.ops.segment_sum(m2.astype(h.dtype), batch, num_segments=G)
    mean = ssum / jnp.maximum(cnt, 1.0)[:, None]
    return jax.nn.log_softmax(mean @ fc_W + fc_b, axis=1)


# speedup vs baseline: 3.9772x; 3.9772x over previous
import functools

import jax
import jax.numpy as jnp
from jax import lax
from jax.experimental import pallas as pl
from jax.experimental.pallas import tpu as pltpu
from jax.experimental.pallas import tpu_sc as plsc

N = 10000
E = 320000
H = 128
G = 64
NPAD = 10240
EPAD = 323584
NW = 32
EPC = EPAD // NW
CH = 128
NT = NPAD // NW
CAP = 16384
NEG = -3.0e38

_MESH = plsc.VectorSubcoreMesh(core_axis_name="c", subcore_axis_name="s")


def _agg_body(table, src, dst, zeros, out, idxs, idxd, rows, acc, sem):
    cid = lax.axis_index("c")
    sid = lax.axis_index("s")
    rps = NPAD // 16
    pltpu.sync_copy(zeros.at[pl.ds(sid * rps, rps)], acc.at[pl.ds(sid * rps, rps)])
    plsc.subcore_barrier()
    base = (sid * 2 + cid) * EPC

    def body(i, _):
        off = base + i * CH
        pltpu.sync_copy(src.at[pl.ds(off, CH)], idxs)
        pltpu.sync_copy(dst.at[pl.ds(off, CH)], idxd)
        pltpu.async_copy(table.at[idxs], rows, sem).wait()
        pltpu.sync_copy(rows, acc.at[idxd], add=True)
        return 0

    lax.fori_loop(0, EPC // CH, body, 0)
    plsc.subcore_barrier()
    pltpu.sync_copy(acc.at[pl.ds(sid * rps, rps)],
                    out.at[cid, pl.ds(sid * rps, rps)])


def _hist_body(dst, zeros, erows, out, idxd, rows, acc, sem):
    cid = lax.axis_index("c")
    sid = lax.axis_index("s")
    rps = NPAD // 16
    pltpu.sync_copy(zeros.at[pl.ds(sid * rps, rps)], acc.at[pl.ds(sid * rps, rps)])
    pltpu.sync_copy(erows, rows)
    plsc.subcore_barrier()
    base = (sid * 2 + cid) * EPC

    def body(i, _):
        pltpu.sync_copy(dst.at[pl.ds(base + i * CH, CH)], idxd)
        pltpu.sync_copy(rows, acc.at[idxd], add=True)
        return 0

    lax.fori_loop(0, EPC // CH, body, 0)
    plsc.subcore_barrier()
    pltpu.sync_copy(acc.at[pl.ds(sid * rps, rps)],
                    out.at[cid, pl.ds(sid * rps, rps)])


_hist_k = pl.kernel(
    _hist_body,
    out_type=jax.ShapeDtypeStruct((2, NPAD, H), jnp.float32),
    mesh=_MESH,
    scratch_types=[
        pltpu.VMEM((CH,), jnp.int32),
        pltpu.VMEM((CH, H), jnp.float32),
        pltpu.VMEM_SHARED((NPAD, H), jnp.float32),
        pltpu.SemaphoreType.DMA,
    ],
)


def _hist(dst_p):
    erows = jnp.zeros((CH, H), jnp.float32).at[:, 0].set(1.0)
    out = _hist_k(dst_p, jnp.zeros((NPAD, H), jnp.float32), erows)
    return out[0] + out[1]


@functools.cache
def _make_agg(w):
    return pl.kernel(
        _agg_body,
        out_type=jax.ShapeDtypeStruct((2, NPAD, w), jnp.float32),
        mesh=_MESH,
        scratch_types=[
            pltpu.VMEM((CH,), jnp.int32),
            pltpu.VMEM((CH,), jnp.int32),
            pltpu.VMEM((CH, w), jnp.float32),
            pltpu.VMEM_SHARED((NPAD, w), jnp.float32),
            pltpu.SemaphoreType.DMA,
        ],
    )


def _agg(table, src, dst):
    w = table.shape[1]
    out = _make_agg(w)(table, src, dst, jnp.zeros((NPAD, w), jnp.float32))
    return out[0] + out[1]


def _ecg_body(ta, tb, dst, src, outa, outb, idxd, idxs, rowsa, rowsb, sem):
    cid = lax.axis_index("c")
    sid = lax.axis_index("s")
    base = (sid * 2 + cid) * EPC

    def body(i, _):
        off = base + i * CH
        pltpu.sync_copy(dst.at[pl.ds(off, CH)], idxd)
        pltpu.sync_copy(src.at[pl.ds(off, CH)], idxs)
        pltpu.async_copy(ta.at[idxd], rowsa, sem).wait()
        pltpu.sync_copy(rowsa, outa.at[pl.ds(off, CH)])
        pltpu.async_copy(tb.at[idxs], rowsb, sem).wait()
        pltpu.sync_copy(rowsb, outb.at[pl.ds(off, CH)])
        return 0

    lax.fori_loop(0, EPC // CH, body, 0)


@functools.cache
def _make_ecg(w):
    return pl.kernel(
        _ecg_body,
        out_type=(jax.ShapeDtypeStruct((EPAD, w), jnp.float32),
                  jax.ShapeDtypeStruct((EPAD, w), jnp.float32)),
        mesh=_MESH,
        scratch_types=[
            pltpu.VMEM((CH,), jnp.int32),
            pltpu.VMEM((CH,), jnp.int32),
            pltpu.VMEM((CH, w), jnp.float32),
            pltpu.VMEM((CH, w), jnp.float32),
            pltpu.SemaphoreType.DMA,
        ],
    )


def _smax_body(m, dst, out, dchunk, eidl, dl, rows, table, sem):
    cid = lax.axis_index("c")
    sid = lax.axis_index("s")
    wid = sid * 2 + cid
    n0 = wid * NT

    iota = lax.iota(jnp.int32, 16)

    def init(i, _):
        plsc.store_scatter(table, [i * 16 + iota], jnp.full((16,), NEG, jnp.float32))
        return 0

    lax.fori_loop(0, (NT * H + H) // 16, init, 0)

    def prefill(i, _):
        plsc.store_scatter(eidl, [i * 16 + iota], jnp.zeros((16,), jnp.int32))
        plsc.store_scatter(dl, [i * 16 + iota], jnp.full((16,), NT, jnp.int32))
        return 0

    lax.fori_loop(0, (CAP + 16) // 16, prefill, 0)

    def scan_chunk(ch, cnt):
        pltpu.sync_copy(dst.at[pl.ds(ch * 2048, 2048)], dchunk)

        def group(g, cnt):
            d = plsc.load_gather(dchunk, [g * 16 + iota])
            eid = (ch * 2048 + g * 16) + iota
            msk = (d >= n0) & (d < n0 + NT)
            inc = jnp.cumsum(msk.astype(jnp.int32))
            pos = cnt + inc - 1
            plsc.store_scatter(eidl, [pos], eid, mask=msk)
            plsc.store_scatter(dl, [pos], d - n0, mask=msk)
            return jnp.minimum(cnt + inc[15], CAP)

        return lax.fori_loop(0, 128, group, cnt)

    cnt = lax.fori_loop(0, EPAD // 2048, scan_chunk, 0)

    def flush(f, _):
        pltpu.async_copy(m.at[eidl.at[pl.ds(f * CH, CH)]], rows, sem).wait()

        def rmw16(rr, _):
            doffs = plsc.load_gather(dl, [f * CH + rr * 16 + iota])
            for r2 in range(16):
                doff = doffs[r2]
                rsel = jnp.full((16,), rr * 16 + r2, jnp.int32)
                for j in range(8):
                    addrs = doff * H + j * 16 + iota
                    cur = plsc.load_gather(table, [addrs])
                    v = plsc.load_gather(rows, [rsel, j * 16 + iota])
                    plsc.store_scatter(table, [addrs], jnp.maximum(cur, v))
            return 0

        lax.fori_loop(0, CH // 16, rmw16, 0)
        return 0

    lax.fori_loop(0, (cnt + CH - 1) // CH, flush, 0)
    pltpu.sync_copy(table.at[pl.ds(0, NT * H)], out.at[pl.ds(n0 * H, NT * H)])


_smax_flat = pl.kernel(
    _smax_body,
    out_type=jax.ShapeDtypeStruct((NPAD * H,), jnp.float32),
    mesh=_MESH,
    compiler_params=pltpu.CompilerParams(needs_layout_passes=False),
    scratch_types=[
        pltpu.VMEM((2048,), jnp.int32),
        pltpu.VMEM((CAP + 16,), jnp.int32),
        pltpu.VMEM((CAP + 16,), jnp.int32),
        pltpu.VMEM((CH, H), jnp.float32),
        pltpu.VMEM((NT * H + H,), jnp.float32),
        pltpu.SemaphoreType.DMA,
    ],
)


def _smax(m, dst_p):
    return _smax_flat(m, dst_p).reshape(NPAD, H)


def _smax_jnp(m, dst_p):
    return jnp.full((NPAD, H), NEG, jnp.float32).at[dst_p].max(m)


def _mm_kernel(x_ref, w_ref, b_ref, o_ref, *, act):
    y = jnp.dot(x_ref[...], w_ref[...], preferred_element_type=jnp.float32)
    y = y + b_ref[...]
    if act == "relu":
        y = jnp.maximum(y, 0.0)
    o_ref[...] = y


def _matmul(x, w, b, act=None, block=512):
    m, k = x.shape
    n = w.shape[1]
    assert m % block == 0
    return pl.pallas_call(
        functools.partial(_mm_kernel, act=act),
        grid=(m // block,),
        in_specs=[
            pl.BlockSpec((block, k), lambda i: (i, 0)),
            pl.BlockSpec((k, n), lambda i: (0, 0)),
            pl.BlockSpec((1, n), lambda i: (0, 0)),
        ],
        out_specs=pl.BlockSpec((block, n), lambda i: (i, 0)),
        out_shape=jax.ShapeDtypeStruct((m, n), jnp.float32),
    )(x, w, b.reshape(1, -1))


def _mme_kernel(a_ref, b_ref, w_ref, o_ref):
    pe = jnp.maximum(a_ref[...] + b_ref[...], 0.0)
    o_ref[...] = jnp.dot(pe, w_ref[...], preferred_element_type=jnp.float32)


def _mme_mask_kernel(a_ref, b_ref, w_ref, o_ref):
    a = a_ref[...]
    b = b_ref[...]
    live = (a[:, 0:1] > -1.0e8) & (b[:, 0:1] > -1.0e8)
    pe = jnp.maximum(a + b, 0.0)
    y = jnp.dot(pe, w_ref[...], preferred_element_type=jnp.float32)
    o_ref[...] = jnp.where(live, y, NEG)


def _edge_mlp(ag, bg, w2, masked=False, block=2048):
    m = ag.shape[0]
    return pl.pallas_call(
        _mme_mask_kernel if masked else _mme_kernel,
        grid=(m // block,),
        in_specs=[
            pl.BlockSpec((block, H), lambda i: (i, 0)),
            pl.BlockSpec((block, H), lambda i: (i, 0)),
            pl.BlockSpec((H, H), lambda i: (0, 0)),
        ],
        out_specs=pl.BlockSpec((block, H), lambda i: (i, 0)),
        out_shape=jax.ShapeDtypeStruct((m, H), jnp.float32),
    )(ag, bg, w2)


def _pad_nodes(x):
    return jnp.pad(x, ((0, NPAD - x.shape[0]), (0, 0)))


def _gcn_sc(x, src_p, dst_p, W, b, m1f=None):
    if m1f is None:
        wdeg = _hist(dst_p)[:N, 0]
        deg = wdeg + 1.0
    else:
        t128 = jnp.zeros((NPAD, H), jnp.float32).at[:N, 0].set(m1f)
        wdeg = _agg(t128, src_p, dst_p)[:N, 0]
        deg = m1f * wdeg + 1.0
    dis = deg ** -0.5
    xw = _matmul(_pad_nodes(x), W, jnp.zeros_like(b))
    scale = dis if m1f is None else dis * m1f
    table = xw * _pad_nodes(scale[:, None])
    acc = _agg(table, src_p, dst_p)[:N]
    out = dis[:, None] * (acc if m1f is None else m1f[:, None] * acc) \
        + (dis ** 2)[:, None] * xw[:N]
    return out + b


def _edge_conv_sc(x, src_p, dst_p, W1, b1, W2, b2, m1f=None):
    w1a, w1b = W1[:H], W1[H:]
    a = _matmul(_pad_nodes(x), w1a - w1b, b1)
    bb = _matmul(_pad_nodes(x), w1b, jnp.zeros_like(b1))
    if m1f is not None:
        off = jnp.pad(-1.0e9 * (1.0 - m1f), (0, NPAD - N))
        a = a.at[:, 0].add(off)
        bb = bb.at[:, 0].add(off)
    ag, bg = _make_ecg(H)(a, bb, dst_p, src_p)
    m = _edge_mlp(ag, bg, W2, masked=m1f is not None)
    mx = _smax(m, dst_p)[:N]
    dead = mx[:, 0:1] < -1.0e37
    return jnp.where(dead, 0.0, mx + b2)


def _select_mask(score, batch, mask, tie):
    n = score.shape[0]
    neg = jnp.where(mask, score, -jnp.inf)
    order = jnp.lexsort((tie, -neg, batch))
    total = jax.ops.segment_sum(jnp.ones((n,), jnp.int32), batch, num_segments=G)
    offs = jnp.cumsum(total) - total
    cnt = jax.ops.segment_sum(mask.astype(jnp.int32), batch, num_segments=G)
    k = (4 * cnt + 4) // 5
    bs = batch[order]
    rank = jnp.arange(n, dtype=jnp.int32) - offs[bs]
    sel = (rank < k[bs]) & mask[order]
    new_mask = jnp.zeros((n,), jnp.bool_).at[order].set(sel)
    pos = jnp.zeros((n,), jnp.int32).at[order].set(jnp.arange(n, dtype=jnp.int32))
    return new_mask, pos


def kernel(x, edge_index, batch, gcn1_W, gcn1_b, ec1_W1, ec1_b1, ec1_W2, ec1_b2,
           pool1_p, gcn2_W, gcn2_b, ec2_W1, ec2_b1, ec2_W2, ec2_b2, pool2_p,
           fc_W, fc_b):
    n = x.shape[0]
    src_p = jnp.pad(edge_index[0], (0, EPAD - E))
    dst_p = jnp.pad(edge_index[1], (0, EPAD - E), constant_values=NPAD - 1)

    h = jax.nn.relu(_gcn_sc(x, src_p, dst_p, gcn1_W, gcn1_b))
    h = jax.nn.relu(_edge_conv_sc(h, src_p, dst_p, ec1_W1, ec1_b1, ec1_W2, ec1_b2))
    s1 = jnp.tanh((h @ pool1_p) / jnp.linalg.norm(pool1_p))
    m0 = jnp.ones((n,), jnp.bool_)
    m1, pos1 = _select_mask(s1, batch, m0, jnp.arange(n, dtype=jnp.int32))
    m1f = m1.astype(jnp.float32)
    h = h * s1[:, None]
    h = jax.nn.relu(_gcn_sc(h, src_p, dst_p, gcn2_W, gcn2_b, m1f))
    h = jax.nn.relu(_edge_conv_sc(h, src_p, dst_p, ec2_W1, ec2_b1, ec2_W2, ec2_b2,
                                  m1f))
    s2 = jnp.tanh((h @ pool2_p) / jnp.linalg.norm(pool2_p))
    m2, _ = _select_mask(s2, batch, m1, pos1)
    h = h * s2[:, None]
    hs = jnp.where(m2[:, None], h, 0.0)
    ssum = jax.ops.segment_sum(hs, batch, num_segments=G)
    cnt = jax.ops.segment_sum(m2.astype(h.dtype), batch, num_segments=G)
    mean = ssum / jnp.maximum(cnt, 1.0)[:, None]
    return jax.nn.log_softmax(mean @ fc_W + fc_b, axis=1)


# speedup vs baseline: 4.0903x; 1.0284x over previous
import functools

import jax
import jax.numpy as jnp
from jax import lax
from jax.experimental import pallas as pl
from jax.experimental.pallas import tpu as pltpu
from jax.experimental.pallas import tpu_sc as plsc

N = 10000
E = 320000
H = 128
G = 64
NPAD = 10240
EPAD = 327680
NW = 32
EPC = EPAD // NW
CH = 128
NT = NPAD // NW
CAP = 16384
NEG = -3.0e38

_MESH = plsc.VectorSubcoreMesh(core_axis_name="c", subcore_axis_name="s")


NCH = EPC // CH


def _agg_body(table, src, dst, zeros, out, is0, is1, id0, id1, id2, id3,
              r0, r1, acc, gsem, ssem, isem):
    cid = lax.axis_index("c")
    sid = lax.axis_index("s")
    rps = NPAD // 16
    pltpu.sync_copy(zeros.at[pl.ds(sid * rps, rps)], acc.at[pl.ds(sid * rps, rps)])
    base = (sid * 2 + cid) * EPC
    IS = [is0, is1]
    ID = [id0, id1, id2, id3]
    R = [r0, r1]
    plsc.subcore_barrier()
    pltpu.sync_copy(src.at[pl.ds(base, CH)], IS[0])
    pltpu.sync_copy(dst.at[pl.ds(base, CH)], ID[0])
    gat = [None] * NCH
    scat = [None] * NCH
    ihs = [None] * NCH
    ihd = [None] * NCH
    gat[0] = pltpu.async_copy(table.at[IS[0]], R[0], gsem)
    if NCH > 1:
        ihs[1] = pltpu.async_copy(src.at[pl.ds(base + CH, CH)], IS[1], isem)
        ihd[1] = pltpu.async_copy(dst.at[pl.ds(base + CH, CH)], ID[1], isem)
    for i in range(1, NCH):
        gat[i - 1].wait()
        scat[i - 1] = pltpu.async_copy(R[(i - 1) % 2], acc.at[ID[(i - 1) % 4]],
                                       ssem, add=True)
        ihs[i].wait()
        ihd[i].wait()
        if i >= 2:
            scat[i - 2].wait()
        gat[i] = pltpu.async_copy(table.at[IS[i % 2]], R[i % 2], gsem)
        if i + 1 < NCH:
            off = base + (i + 1) * CH
            ihs[i + 1] = pltpu.async_copy(src.at[pl.ds(off, CH)],
                                          IS[(i + 1) % 2], isem)
            ihd[i + 1] = pltpu.async_copy(dst.at[pl.ds(off, CH)],
                                          ID[(i + 1) % 4], isem)
    gat[NCH - 1].wait()
    scat[NCH - 1] = pltpu.async_copy(R[(NCH - 1) % 2], acc.at[ID[(NCH - 1) % 4]],
                                     ssem, add=True)
    scat[NCH - 2].wait()
    scat[NCH - 1].wait()
    plsc.subcore_barrier()
    pltpu.sync_copy(acc.at[pl.ds(sid * rps, rps)],
                    out.at[cid, pl.ds(sid * rps, rps)])


def _hist_body(dst2, zeros, erows, out, idxd, rows, acc, ssem):
    cid = lax.axis_index("c")
    sid = lax.axis_index("s")
    rps = NPAD // 16
    pltpu.sync_copy(zeros.at[pl.ds(sid * rps, rps)], acc.at[pl.ds(sid * rps, rps)])
    pltpu.sync_copy(erows, rows)
    cb = (sid * 2 + cid) * NCH
    pltpu.sync_copy(dst2.at[pl.ds(cb, NCH)], idxd)
    plsc.subcore_barrier()
    scat = [pltpu.async_copy(rows, acc.at[idxd.at[i]], ssem, add=True)
            for i in range(NCH)]
    for h in scat:
        h.wait()
    plsc.subcore_barrier()
    pltpu.sync_copy(acc.at[pl.ds(sid * rps, rps)],
                    out.at[cid, pl.ds(sid * rps, rps)])


_hist_k = pl.kernel(
    _hist_body,
    out_type=jax.ShapeDtypeStruct((2, NPAD, H), jnp.float32),
    mesh=_MESH,
    scratch_types=[
        pltpu.VMEM((NCH, CH), jnp.int32),
        pltpu.VMEM((CH, H), jnp.float32),
        pltpu.VMEM_SHARED((NPAD, H), jnp.float32),
        pltpu.SemaphoreType.DMA,
    ],
)


def _hist(dst_p):
    erows = jnp.zeros((CH, H), jnp.float32).at[:, 0].set(1.0)
    out = _hist_k(dst_p.reshape(-1, CH), jnp.zeros((NPAD, H), jnp.float32), erows)
    return out[0] + out[1]


@functools.cache
def _make_agg(w):
    return pl.kernel(
        _agg_body,
        out_type=jax.ShapeDtypeStruct((2, NPAD, w), jnp.float32),
        mesh=_MESH,
        scratch_types=(
            [pltpu.VMEM((CH,), jnp.int32) for _ in range(6)]
            + [pltpu.VMEM((CH, w), jnp.float32) for _ in range(2)]
            + [pltpu.VMEM_SHARED((NPAD, w), jnp.float32),
               pltpu.SemaphoreType.DMA,
               pltpu.SemaphoreType.DMA,
               pltpu.SemaphoreType.DMA]
        ),
    )


def _agg(table, src, dst):
    w = table.shape[1]
    out = _make_agg(w)(table, src, dst, jnp.zeros((NPAD, w), jnp.float32))
    return out[0] + out[1]


def _ecg_body(ta, tb, dst2, src2, outa, outb, idxd, idxs, rowsa, rowsb,
              gsem, wsem):
    cid = lax.axis_index("c")
    sid = lax.axis_index("s")
    wid = sid * 2 + cid
    cb = wid * NCH
    base = wid * EPC
    pltpu.sync_copy(dst2.at[pl.ds(cb, NCH)], idxd)
    pltpu.sync_copy(src2.at[pl.ds(cb, NCH)], idxs)
    ga = [None] * NCH
    gb = [None] * NCH
    wa = [None] * NCH
    wb = [None] * NCH
    for i in range(NCH):
        b = i & 1
        if i >= 2:
            wa[i - 2].wait()
            wb[i - 2].wait()
        ga[i] = pltpu.async_copy(ta.at[idxd.at[i]], rowsa.at[b], gsem)
        gb[i] = pltpu.async_copy(tb.at[idxs.at[i]], rowsb.at[b], gsem)
        if i >= 1:
            ga[i - 1].wait()
            gb[i - 1].wait()
            off = base + (i - 1) * CH
            wa[i - 1] = pltpu.async_copy(rowsa.at[(i - 1) & 1],
                                         outa.at[pl.ds(off, CH)], wsem)
            wb[i - 1] = pltpu.async_copy(rowsb.at[(i - 1) & 1],
                                         outb.at[pl.ds(off, CH)], wsem)
    ga[NCH - 1].wait()
    gb[NCH - 1].wait()
    off = base + (NCH - 1) * CH
    wa[NCH - 1] = pltpu.async_copy(rowsa.at[(NCH - 1) & 1],
                                   outa.at[pl.ds(off, CH)], wsem)
    wb[NCH - 1] = pltpu.async_copy(rowsb.at[(NCH - 1) & 1],
                                   outb.at[pl.ds(off, CH)], wsem)
    for i in (NCH - 2, NCH - 1):
        wa[i].wait()
        wb[i].wait()


@functools.cache
def _make_ecg(w):
    return pl.kernel(
        _ecg_body,
        out_type=(jax.ShapeDtypeStruct((EPAD, w), jnp.float32),
                  jax.ShapeDtypeStruct((EPAD, w), jnp.float32)),
        mesh=_MESH,
        scratch_types=[
            pltpu.VMEM((NCH, CH), jnp.int32),
            pltpu.VMEM((NCH, CH), jnp.int32),
            pltpu.VMEM((2, CH, w), jnp.float32),
            pltpu.VMEM((2, CH, w), jnp.float32),
            pltpu.SemaphoreType.DMA,
            pltpu.SemaphoreType.DMA,
        ],
    )


def _smax_body(m, dst, out, dchunk, eidl, dl, rows, table, sem, gsem):
    cid = lax.axis_index("c")
    sid = lax.axis_index("s")
    wid = sid * 2 + cid
    n0 = wid * NT

    iota = lax.iota(jnp.int32, 16)

    def init(i, _):
        plsc.store_scatter(table, [i * 16 + iota], jnp.full((16,), NEG, jnp.float32))
        return 0

    lax.fori_loop(0, (NT * H + H) // 16, init, 0)

    def prefill(i, _):
        plsc.store_scatter(eidl, [i * 16 + iota], jnp.zeros((16,), jnp.int32))
        plsc.store_scatter(dl, [i * 16 + iota], jnp.full((16,), NT, jnp.int32))
        return 0

    lax.fori_loop(0, (CAP + 16) // 16, prefill, 0)

    NSC = EPAD // 2048
    pltpu.async_copy(dst.at[pl.ds(0, 2048)], dchunk.at[pl.ds(0, 2048)], sem)

    def scan_chunk(ch, cnt):
        bcur = lax.bitwise_and(ch, 1)
        pltpu.make_async_copy(dst.at[pl.ds(0, 2048)],
                              dchunk.at[pl.ds(0, 2048)], sem).wait()

        @pl.when(ch + 1 < NSC)
        def _():
            pltpu.async_copy(dst.at[pl.ds((ch + 1) * 2048, 2048)],
                             dchunk.at[pl.ds((1 - bcur) * 2048, 2048)], sem)

        def group(g, cnt):
            d = plsc.load_gather(dchunk, [bcur * 2048 + g * 16 + iota])
            eid = (ch * 2048 + g * 16) + iota
            msk = (d >= n0) & (d < n0 + NT)
            inc = jnp.cumsum(msk.astype(jnp.int32))
            pos = cnt + inc - 1
            plsc.store_scatter(eidl, [pos], eid, mask=msk)
            plsc.store_scatter(dl, [pos], d - n0, mask=msk)
            return jnp.minimum(cnt + inc[15], CAP)

        return lax.fori_loop(0, 128, group, cnt)

    cnt = lax.fori_loop(0, NSC, scan_chunk, 0)
    nf = (cnt + CH - 1) // CH

    @pl.when(nf > 0)
    def _():
        pltpu.async_copy(m.at[eidl.at[pl.ds(0, CH)]], rows.at[pl.ds(0, CH)], gsem)

    def flush(f, _):
        bcur = lax.bitwise_and(f, 1)
        pltpu.make_async_copy(m.at[eidl.at[pl.ds(0, CH)]],
                              rows.at[pl.ds(0, CH)], gsem).wait()

        @pl.when(f + 1 < nf)
        def _():
            pltpu.async_copy(m.at[eidl.at[pl.ds((f + 1) * CH, CH)]],
                             rows.at[pl.ds((1 - bcur) * CH, CH)], gsem)

        def rmw16(rr, _):
            doffs = plsc.load_gather(dl, [f * CH + rr * 16 + iota])
            for r2 in range(16):
                doff = doffs[r2]
                rsel = jnp.full((16,), 1, jnp.int32) * (bcur * CH + rr * 16 + r2)
                for j in range(8):
                    addrs = doff * H + j * 16 + iota
                    cur = plsc.load_gather(table, [addrs])
                    v = plsc.load_gather(rows, [rsel, j * 16 + iota])
                    plsc.store_scatter(table, [addrs], jnp.maximum(cur, v))
            return 0

        lax.fori_loop(0, CH // 16, rmw16, 0)
        return 0

    lax.fori_loop(0, nf, flush, 0)
    pltpu.sync_copy(table.at[pl.ds(0, NT * H)], out.at[pl.ds(n0 * H, NT * H)])


_smax_flat = pl.kernel(
    _smax_body,
    out_type=jax.ShapeDtypeStruct((NPAD * H,), jnp.float32),
    mesh=_MESH,
    compiler_params=pltpu.CompilerParams(needs_layout_passes=False),
    scratch_types=[
        pltpu.VMEM((2 * 2048,), jnp.int32),
        pltpu.VMEM((CAP + 16,), jnp.int32),
        pltpu.VMEM((CAP + 16,), jnp.int32),
        pltpu.VMEM((2 * CH, H), jnp.float32),
        pltpu.VMEM((NT * H + H,), jnp.float32),
        pltpu.SemaphoreType.DMA,
        pltpu.SemaphoreType.DMA,
    ],
)


def _smax(m, dst_p):
    return _smax_flat(m, dst_p).reshape(NPAD, H)


def _smax_jnp(m, dst_p):
    return jnp.full((NPAD, H), NEG, jnp.float32).at[dst_p].max(m)


def _mm_kernel(x_ref, w_ref, b_ref, o_ref, *, act):
    y = jnp.dot(x_ref[...], w_ref[...], preferred_element_type=jnp.float32)
    y = y + b_ref[...]
    if act == "relu":
        y = jnp.maximum(y, 0.0)
    o_ref[...] = y


def _matmul(x, w, b, act=None, block=512):
    m, k = x.shape
    n = w.shape[1]
    assert m % block == 0
    return pl.pallas_call(
        functools.partial(_mm_kernel, act=act),
        grid=(m // block,),
        in_specs=[
            pl.BlockSpec((block, k), lambda i: (i, 0)),
            pl.BlockSpec((k, n), lambda i: (0, 0)),
            pl.BlockSpec((1, n), lambda i: (0, 0)),
        ],
        out_specs=pl.BlockSpec((block, n), lambda i: (i, 0)),
        out_shape=jax.ShapeDtypeStruct((m, n), jnp.float32),
    )(x, w, b.reshape(1, -1))


def _mme_kernel(a_ref, b_ref, w_ref, o_ref):
    pe = jnp.maximum(a_ref[...] + b_ref[...], 0.0)
    o_ref[...] = jnp.dot(pe, w_ref[...], preferred_element_type=jnp.float32)


def _mme_mask_kernel(a_ref, b_ref, w_ref, o_ref):
    a = a_ref[...]
    b = b_ref[...]
    live = (a[:, 0:1] > -1.0e8) & (b[:, 0:1] > -1.0e8)
    pe = jnp.maximum(a + b, 0.0)
    y = jnp.dot(pe, w_ref[...], preferred_element_type=jnp.float32)
    o_ref[...] = jnp.where(live, y, NEG)


def _edge_mlp(ag, bg, w2, masked=False, block=2048):
    m = ag.shape[0]
    return pl.pallas_call(
        _mme_mask_kernel if masked else _mme_kernel,
        grid=(m // block,),
        in_specs=[
            pl.BlockSpec((block, H), lambda i: (i, 0)),
            pl.BlockSpec((block, H), lambda i: (i, 0)),
            pl.BlockSpec((H, H), lambda i: (0, 0)),
        ],
        out_specs=pl.BlockSpec((block, H), lambda i: (i, 0)),
        out_shape=jax.ShapeDtypeStruct((m, H), jnp.float32),
    )(ag, bg, w2)


def _pad_nodes(x):
    return jnp.pad(x, ((0, NPAD - x.shape[0]), (0, 0)))


def _gcn_sc(x, src_p, dst_p, W, b, m1f=None):
    if m1f is None:
        wdeg = _hist(dst_p)[:N, 0]
        deg = wdeg + 1.0
    else:
        t128 = jnp.zeros((NPAD, H), jnp.float32).at[:N, 0].set(m1f)
        wdeg = _agg(t128, src_p, dst_p)[:N, 0]
        deg = m1f * wdeg + 1.0
    dis = deg ** -0.5
    xw = _matmul(_pad_nodes(x), W, jnp.zeros_like(b))
    scale = dis if m1f is None else dis * m1f
    table = xw * _pad_nodes(scale[:, None])
    acc = _agg(table, src_p, dst_p)[:N]
    out = dis[:, None] * (acc if m1f is None else m1f[:, None] * acc) \
        + (dis ** 2)[:, None] * xw[:N]
    return out + b


def _edge_conv_sc(x, src_p, dst_p, W1, b1, W2, b2, m1f=None):
    w1a, w1b = W1[:H], W1[H:]
    a = _matmul(_pad_nodes(x), w1a - w1b, b1)
    bb = _matmul(_pad_nodes(x), w1b, jnp.zeros_like(b1))
    if m1f is not None:
        off = jnp.pad(-1.0e9 * (1.0 - m1f), (0, NPAD - N))
        a = a.at[:, 0].add(off)
        bb = bb.at[:, 0].add(off)
    ag, bg = _make_ecg(H)(a, bb, dst_p.reshape(-1, CH), src_p.reshape(-1, CH))
    m = _edge_mlp(ag, bg, W2, masked=m1f is not None)
    mx = _smax(m, dst_p)[:N]
    dead = mx[:, 0:1] < -1.0e37
    return jnp.where(dead, 0.0, mx + b2)


def _select_mask(score, batch, mask, tie):
    n = score.shape[0]
    neg = jnp.where(mask, score, -jnp.inf)
    order = jnp.lexsort((tie, -neg, batch))
    total = jax.ops.segment_sum(jnp.ones((n,), jnp.int32), batch, num_segments=G)
    offs = jnp.cumsum(total) - total
    cnt = jax.ops.segment_sum(mask.astype(jnp.int32), batch, num_segments=G)
    k = (4 * cnt + 4) // 5
    bs = batch[order]
    rank = jnp.arange(n, dtype=jnp.int32) - offs[bs]
    sel = (rank < k[bs]) & mask[order]
    new_mask = jnp.zeros((n,), jnp.bool_).at[order].set(sel)
    pos = jnp.zeros((n,), jnp.int32).at[order].set(jnp.arange(n, dtype=jnp.int32))
    return new_mask, pos


def kernel(x, edge_index, batch, gcn1_W, gcn1_b, ec1_W1, ec1_b1, ec1_W2, ec1_b2,
           pool1_p, gcn2_W, gcn2_b, ec2_W1, ec2_b1, ec2_W2, ec2_b2, pool2_p,
           fc_W, fc_b):
    n = x.shape[0]
    src_p = jnp.pad(edge_index[0], (0, EPAD - E))
    dst_p = jnp.pad(edge_index[1], (0, EPAD - E), constant_values=NPAD - 1)

    h = jax.nn.relu(_gcn_sc(x, src_p, dst_p, gcn1_W, gcn1_b))
    h = jax.nn.relu(_edge_conv_sc(h, src_p, dst_p, ec1_W1, ec1_b1, ec1_W2, ec1_b2))
    s1 = jnp.tanh((h @ pool1_p) / jnp.linalg.norm(pool1_p))
    m0 = jnp.ones((n,), jnp.bool_)
    m1, pos1 = _select_mask(s1, batch, m0, jnp.arange(n, dtype=jnp.int32))
    m1f = m1.astype(jnp.float32)
    h = h * s1[:, None]
    h = jax.nn.relu(_gcn_sc(h, src_p, dst_p, gcn2_W, gcn2_b, m1f))
    h = jax.nn.relu(_edge_conv_sc(h, src_p, dst_p, ec2_W1, ec2_b1, ec2_W2, ec2_b2,
                                  m1f))
    s2 = jnp.tanh((h @ pool2_p) / jnp.linalg.norm(pool2_p))
    m2, _ = _select_mask(s2, batch, m1, pos1)
    h = h * s2[:, None]
    hs = jnp.where(m2[:, None], h, 0.0)
    ssum = jax.ops.segment_sum(hs, batch, num_segments=G)
    cnt = jax.ops.segment_sum(m2.astype(h.dtype), batch, num_segments=G)
    mean = ssum / jnp.maximum(cnt, 1.0)[:, None]
    return jax.nn.log_softmax(mean @ fc_W + fc_b, axis=1)
